# Initial kernel scaffold; baseline (speedup 1.0000x reference)
import jax
import jax.numpy as jnp
from jax.experimental import pallas as pl


def kernel(rgb_local, ir_local, W_gate_rgb, b_gate_rgb, W_gate_ir, b_gate_ir, W_exp_rgb, b_exp_rgb, W_exp_ir, b_exp_ir):
    raise NotImplementedError("write your pallas kernel here")


# speedup vs baseline: 1.7851x; 1.7851x over previous
import functools

import jax
import jax.numpy as jnp
from jax.experimental import pallas as pl
from jax.experimental.pallas import tpu as pltpu

D = 2048
BJ = 512
NJ = D // BJ


def _gate_kernel(x_ref, wgr_ref, bgr_ref, wgi_ref, bgi_ref,
                 idx_ref, probs_ref, mir_ref, mii_ref):
    x = x_ref[...]
    dn = (((1,), (1,)), ((), ()))
    sr = jax.nn.relu(
        jax.lax.dot_general(x, wgr_ref[...], dn,
                            preferred_element_type=jnp.float32) + bgr_ref[...])
    si = jax.nn.relu(
        jax.lax.dot_general(x, wgi_ref[...], dn,
                            preferred_element_type=jnp.float32) + bgi_ref[...])

    def route(s):
        a, b, c, d = s[:, 0:1], s[:, 1:2], s[:, 2:3], s[:, 3:4]
        i1 = jnp.where(a >= b, 0, 1).astype(jnp.int32)
        s1 = jnp.maximum(a, b)
        i2 = jnp.where(c >= d, 2, 3).astype(jnp.int32)
        s2 = jnp.maximum(c, d)
        m = jnp.maximum(s1, s2)
        e1 = jnp.exp(s1 - m)
        e2 = jnp.exp(s2 - m)
        denom = e1 + e2
        p1 = e1 / denom
        p2 = e2 / denom
        mi = jnp.where(p1 >= p2, 0, 1).astype(jnp.int32)
        return i1, i2, p1, p2, mi

    ir1, ir2, pr1, pr2, mir = route(sr)
    ii1, ii2, pi1, pi2, mii = route(si)

    idx_ref[...] = jnp.concatenate([ir1, ir2, ii1, ii2], axis=1)
    probs_ref[...] = jnp.concatenate([pr1, pr2, pi1, pi2], axis=1)
    mir_ref[...] = mir
    mii_ref[...] = mii


def _combine_kernel(idx_ref, probs_ref, x_ref, wr_ref, wi_ref,
                    br_ref, bi_ref, out_ref):
    k = pl.program_id(1)
    pr = probs_ref[k]
    pi = probs_ref[2 + k]
    x = x_ref[...]
    dn = (((1,), (1,)), ((), ()))
    yr = jax.lax.dot_general(x, wr_ref[0], dn,
                             preferred_element_type=jnp.float32)
    yi = jax.lax.dot_general(x, wi_ref[0], dn,
                             preferred_element_type=jnp.float32)
    contrib = pr * (yr + br_ref[0]) + pi * (yi + bi_ref[0])

    @pl.when(k == 0)
    def _init():
        out_ref[...] = contrib

    @pl.when(k == 1)
    def _acc():
        out_ref[...] += contrib


@jax.jit
def kernel(rgb_local, ir_local, W_gate_rgb, b_gate_rgb, W_gate_ir, b_gate_ir,
           W_exp_rgb, b_exp_rgb, W_exp_ir, b_exp_ir):
    B = rgb_local.shape[0]
    x = jnp.concatenate(
        [rgb_local.reshape(B, -1), ir_local.reshape(B, -1)], axis=1)

    idx, probs, max_idx_rgb, max_idx_ir = pl.pallas_call(
        _gate_kernel,
        out_shape=(
            jax.ShapeDtypeStruct((1, 4), jnp.int32),
            jax.ShapeDtypeStruct((1, 4), jnp.float32),
            jax.ShapeDtypeStruct((1, 1), jnp.int32),
            jax.ShapeDtypeStruct((1, 1), jnp.int32),
        ),
    )(x, W_gate_rgb, b_gate_rgb.reshape(1, 4), W_gate_ir,
      b_gate_ir.reshape(1, 4))

    grid_spec = pltpu.PrefetchScalarGridSpec(
        num_scalar_prefetch=2,
        grid=(NJ, 2),
        in_specs=[
            pl.BlockSpec((1, D), lambda j, k, idx, p: (0, 0)),
            pl.BlockSpec((1, BJ, D), lambda j, k, idx, p: (idx[k], j, 0)),
            pl.BlockSpec((1, BJ, D), lambda j, k, idx, p: (idx[2 + k], j, 0)),
            pl.BlockSpec((1, 1, BJ), lambda j, k, idx, p: (idx[k], 0, j)),
            pl.BlockSpec((1, 1, BJ), lambda j, k, idx, p: (idx[2 + k], 0, j)),
        ],
        out_specs=pl.BlockSpec((1, BJ), lambda j, k, idx, p: (0, j)),
    )
    combined = pl.pallas_call(
        _combine_kernel,
        grid_spec=grid_spec,
        out_shape=jax.ShapeDtypeStruct((1, D), jnp.float32),
        compiler_params=pltpu.CompilerParams(
            dimension_semantics=("arbitrary", "arbitrary")),
    )(idx.reshape(4), probs.reshape(4), x, W_exp_rgb, W_exp_ir,
      b_exp_rgb.reshape(4, 1, D), b_exp_ir.reshape(4, 1, D))

    return (combined, max_idx_rgb.reshape(1), max_idx_ir.reshape(1))
